# baseline (speedup 1.0000x reference)
import jax
import jax.numpy as jnp
from jax.experimental import pallas as pl

_RADIUS2 = 1.0
_CC = 128


def _bq_kernel(xyz_ref, xyzt_ref, out_ref):
    n = xyzt_ref.shape[2]
    x0q = xyzt_ref[0, 0, :][None, :]
    x1q = xyzt_ref[0, 1, :][None, :]
    x2q = xyzt_ref[0, 2, :][None, :]
    sq_q = x0q * x0q + x1q * x1q + x2q * x2q
    qT = xyzt_ref[0].astype(jnp.bfloat16)

    def body(state):
        k, best = state
        c = k * _CC
        xc = xyz_ref[0, pl.ds(c, _CC), :]
        x0c = xc[:, 0][:, None]
        x1c = xc[:, 1][:, None]
        x2c = xc[:, 2][:, None]
        sq_c = x0c * x0c + x1c * x1c + x2c * x2c
        dot = jax.lax.dot_general(
            xc.astype(jnp.bfloat16), qT,
            (((1,), (0,)), ((), ())),
            preferred_element_type=jnp.float32)
        d2 = (sq_q + sq_c) - 2.0 * dot
        mask = d2 < _RADIUS2
        col = jax.lax.broadcasted_iota(jnp.int32, (_CC, 1), 0) + c
        enc = jnp.where(mask, col, n)
        best = jnp.minimum(best, jnp.min(enc, axis=0, keepdims=True))
        return (k + 1, best)

    def cond(state):
        k, best = state
        return jnp.logical_and(k * _CC < n, jnp.max(best) == n)

    init = (jnp.int32(0), jnp.full((1, n), n, jnp.int32))
    _, best = jax.lax.while_loop(cond, body, init)
    best = jnp.where(best == n, 0, best)
    out_ref[0] = best


def kernel(p):
    b, n, _ = p.shape
    xyz = p[:, :, 0:3]
    xyzt = jnp.transpose(xyz, (0, 2, 1))
    out = pl.pallas_call(
        _bq_kernel,
        grid=(b,),
        in_specs=[
            pl.BlockSpec((1, n, 3), lambda bi: (bi, 0, 0)),
            pl.BlockSpec((1, 3, n), lambda bi: (bi, 0, 0)),
        ],
        out_specs=pl.BlockSpec((1, 1, n), lambda bi: (bi, 0, 0)),
        out_shape=jax.ShapeDtypeStruct((b, 1, n), jnp.int32),
    )(xyz, xyzt)
    return out.reshape(b, n, 1)


# speedup vs baseline: 1.0124x; 1.0124x over previous
import jax
import jax.numpy as jnp
from jax.experimental import pallas as pl

_RADIUS2 = 1.0
_CC = 128


def _bq_kernel(xyz_ref, xyzt_ref, out_ref):
    nb = xyz_ref.shape[0]
    n = xyzt_ref.shape[2]
    sq_q = []
    qt16 = []
    for bi in range(nb):
        x0q = xyzt_ref[bi, 0, :][None, :]
        x1q = xyzt_ref[bi, 1, :][None, :]
        x2q = xyzt_ref[bi, 2, :][None, :]
        sq_q.append(x0q * x0q + x1q * x1q + x2q * x2q)
        qt16.append(xyzt_ref[bi].astype(jnp.bfloat16))
    col_base = jax.lax.broadcasted_iota(jnp.int32, (_CC, 1), 0)

    def body(state):
        k = state[0]
        c = k * _CC
        col = col_base + c
        best = []
        for bi in range(nb):
            xc = xyz_ref[bi, pl.ds(c, _CC), :]
            x0c = xc[:, 0][:, None]
            x1c = xc[:, 1][:, None]
            x2c = xc[:, 2][:, None]
            sq_c = x0c * x0c + x1c * x1c + x2c * x2c
            dot = jax.lax.dot_general(
                xc.astype(jnp.bfloat16), qt16[bi],
                (((1,), (0,)), ((), ())),
                preferred_element_type=jnp.float32)
            d2 = (sq_q[bi] + sq_c) - 2.0 * dot
            enc = jnp.where(d2 < _RADIUS2, col, n)
            best.append(jnp.minimum(state[1 + bi],
                                    jnp.min(enc, axis=0, keepdims=True)))
        return (k + 1, *best)

    def cond(state):
        unfound = state[1] == n
        for bi in range(1, nb):
            unfound = jnp.logical_or(unfound, state[1 + bi] == n)
        return jnp.logical_and(state[0] * _CC < n, jnp.any(unfound))

    init = (jnp.int32(0),) + tuple(
        jnp.full((1, n), n, jnp.int32) for _ in range(nb))
    final = jax.lax.while_loop(cond, body, init)
    for bi in range(nb):
        best = final[1 + bi]
        out_ref[bi] = jnp.where(best == n, 0, best)


def kernel(p):
    b, n, _ = p.shape
    xyz = p[:, :, 0:3]
    xyzt = jnp.transpose(xyz, (0, 2, 1))
    out = pl.pallas_call(
        _bq_kernel,
        in_specs=[
            pl.BlockSpec((b, n, 3), lambda: (0, 0, 0)),
            pl.BlockSpec((b, 3, n), lambda: (0, 0, 0)),
        ],
        out_specs=pl.BlockSpec((b, 1, n), lambda: (0, 0, 0)),
        out_shape=jax.ShapeDtypeStruct((b, 1, n), jnp.int32),
    )(xyz, xyzt)
    return out.reshape(b, n, 1)
